# baseline (speedup 1.0000x reference)
import functools

import jax
import jax.numpy as jnp
from jax import lax
from jax.experimental import pallas as pl
from jax.experimental.pallas import tpu as pltpu
from jax.experimental.pallas import tpu_sc as plsc

_D = 128
_B = 320000
_NC = 2
_NS = 16
_NW = _NC * _NS
_BPW = _B // _NW
_C = 80
_NCH = _BPW // _C
_NBUF = 5
_F = 4
_NG = _NCH // _NBUF

_mesh = plsc.VectorSubcoreMesh(core_axis_name="c", subcore_axis_name="s")


@functools.partial(
    pl.kernel,
    out_type=jax.ShapeDtypeStruct((_B, _D), jnp.float32),
    mesh=_mesh,
    scratch_types=(
        [pltpu.VMEM((_NCH, _C), jnp.int32)]
        + [pltpu.VMEM((_C, _D), jnp.float32) for _ in range(_NBUF)]
        + [pltpu.SemaphoreType.DMA for _ in range(2 * _NBUF)]
    ),
)
def _pe_gather(idx_hbm, table_hbm, out_hbm, idx_v, *bufs_and_sems):
    bufs = bufs_and_sems[:_NBUF]
    sin = bufs_and_sems[_NBUF : 2 * _NBUF]
    sout = bufs_and_sems[2 * _NBUF :]

    wid = lax.axis_index("s") * _NC + lax.axis_index("c")
    base = wid * _BPW
    pltpu.sync_copy(idx_hbm.at[wid], idx_v)

    def gather(j, b):
        pltpu.async_copy(table_hbm.at[idx_v.at[j]], bufs[b], sin[b])

    def wait_gather(b):
        pltpu.make_async_copy(out_hbm.at[pl.ds(0, _C)], bufs[b], sin[b]).wait()

    def writeback(j, b):
        pltpu.async_copy(bufs[b], out_hbm.at[pl.ds(base + j * _C, _C)], sout[b])

    def wait_writeback(b):
        pltpu.make_async_copy(bufs[b], out_hbm.at[pl.ds(0, _C)], sout[b]).wait()

    for b in range(_F):
        gather(b, b)

    for b in range(_NBUF):
        jp = b + _F
        if jp < _NBUF:
            gather(jp, jp)
        else:
            bp = jp % _NBUF
            wait_writeback(bp)
            gather(jp, bp)
        wait_gather(b)
        writeback(b, b)

    def group(g, carry):
        for b in range(_NBUF):
            j = g * _NBUF + b
            bp = (b + _F) % _NBUF
            wait_writeback(bp)
            gather(j + _F, bp)
            wait_gather(b)
            writeback(j, b)
        return carry

    lax.fori_loop(1, _NG - 1, group, 0)

    for b in range(_NBUF):
        j = (_NG - 1) * _NBUF + b
        jp = j + _F
        if jp < _NCH:
            bp = (b + _F) % _NBUF
            wait_writeback(bp)
            gather(jp, bp)
        wait_gather(b)
        writeback(j, b)

    for b in range(_NBUF):
        wait_writeback(b)


def kernel(edge_type, pe):
    n_rows = pe.shape[0]
    pe_rep = jnp.tile(pe, (_NW, 1))
    idx3 = edge_type.astype(jnp.int32).reshape(_NW, _NCH, _C)
    idx3 = idx3 + (jnp.arange(_NW, dtype=jnp.int32) * n_rows)[:, None, None]
    return _pe_gather(idx3, pe_rep)


# speedup vs baseline: 1.9670x; 1.9670x over previous
import functools

import jax
import jax.numpy as jnp
from jax import lax
from jax.experimental import pallas as pl
from jax.experimental.pallas import tpu as pltpu
from jax.experimental.pallas import tpu_sc as plsc

_D = 128
_V = 100
_B = 320000
_NC = 2
_NS = 16
_NW = _NC * _NS
_BPW = _B // _NW
_C = 80
_NCH = _BPW // _C
_NBUF = 5
_F = 3
_NG = _NCH // _NBUF

_mesh = plsc.VectorSubcoreMesh(core_axis_name="c", subcore_axis_name="s")


@functools.partial(
    pl.kernel,
    out_type=jax.ShapeDtypeStruct((_B, _D), jnp.float32),
    mesh=_mesh,
    scratch_types=(
        [
            pltpu.VMEM((_NCH, _C), jnp.int32),
            pltpu.VMEM_SHARED((_V, _D), jnp.float32),
            pltpu.VMEM((_V, _D), jnp.float32),
        ]
        + [pltpu.VMEM((_C, _D), jnp.float32) for _ in range(_NBUF)]
        + [pltpu.SemaphoreType.DMA for _ in range(2 * _NBUF)]
    ),
)
def _pe_gather(idx_hbm, table_hbm, out_hbm, idx_v, table_sh, table_tmp,
               *bufs_and_sems):
    bufs = bufs_and_sems[:_NBUF]
    sin = bufs_and_sems[_NBUF : 2 * _NBUF]
    sout = bufs_and_sems[2 * _NBUF :]

    sid = lax.axis_index("s")
    wid = sid * _NC + lax.axis_index("c")
    base = wid * _BPW

    @pl.when(sid == 0)
    def _():
        pltpu.sync_copy(table_hbm, table_tmp)
        pltpu.sync_copy(table_tmp, table_sh)

    pltpu.sync_copy(idx_hbm.at[wid], idx_v)
    plsc.subcore_barrier()

    def gather(j, b):
        pltpu.async_copy(table_sh.at[idx_v.at[j]], bufs[b], sin[b])

    def wait_gather(b):
        pltpu.make_async_copy(out_hbm.at[pl.ds(0, _C)], bufs[b], sin[b]).wait()

    def writeback(j, b):
        pltpu.async_copy(bufs[b], out_hbm.at[pl.ds(base + j * _C, _C)], sout[b])

    def wait_writeback(b):
        pltpu.make_async_copy(bufs[b], out_hbm.at[pl.ds(0, _C)], sout[b]).wait()

    for b in range(_F):
        gather(b, b)

    for b in range(_NBUF):
        jp = b + _F
        if jp < _NBUF:
            gather(jp, jp)
        else:
            bp = jp % _NBUF
            wait_writeback(bp)
            gather(jp, bp)
        wait_gather(b)
        writeback(b, b)

    def group(g, carry):
        for b in range(_NBUF):
            j = g * _NBUF + b
            bp = (b + _F) % _NBUF
            wait_writeback(bp)
            gather(j + _F, bp)
            wait_gather(b)
            writeback(j, b)
        return carry

    lax.fori_loop(1, _NG - 1, group, 0)

    for b in range(_NBUF):
        j = (_NG - 1) * _NBUF + b
        jp = j + _F
        if jp < _NCH:
            bp = (b + _F) % _NBUF
            wait_writeback(bp)
            gather(jp, bp)
        wait_gather(b)
        writeback(j, b)

    for b in range(_NBUF):
        wait_writeback(b)


def kernel(edge_type, pe):
    idx3 = edge_type.astype(jnp.int32).reshape(_NW, _NCH, _C)
    return _pe_gather(idx3, pe)
